# baseline (speedup 1.0000x reference)
import functools

import jax
import jax.numpy as jnp
from jax import lax
from jax.experimental import pallas as pl
from jax.experimental.pallas import tpu as pltpu
from jax.experimental.pallas import tpu_sc as plsc

_T = 0.05
_MOM = 0.5
_LOG2E = 1.4426950408889634
_SSQ_CLAMP = 1e-6

_NC = 2
_NS = 16
_NW = _NC * _NS
_LANE = 128

_HBM = pltpu.MemorySpace.HBM
_VMEM = pltpu.MemorySpace.VMEM
_SMEM = pltpu.MemorySpace.SMEM


def _sc_gather(m1, m2, idx_flat, total):
    d = m1.shape[1]
    n_chunks = total // _LANE
    per_worker = n_chunks // _NW
    mesh = plsc.VectorSubcoreMesh(
        core_axis_name="c", subcore_axis_name="s",
        num_cores=_NC, num_subcores=_NS)

    @functools.partial(
        pl.kernel,
        out_type=jax.ShapeDtypeStruct((2, total, d), jnp.float32),
        mesh=mesh,
        scratch_types=[
            pltpu.VMEM((_LANE,), jnp.int32),
            pltpu.VMEM((_LANE, d), jnp.float32),
            pltpu.SemaphoreType.DMA,
        ],
    )
    def gather_kernel(m1_hbm, m2_hbm, idx_hbm, w_hbm, idx_v, rows_v, sem):
        wid = lax.axis_index("s") * _NC + lax.axis_index("c")
        for t in range(per_worker):
            chunk = wid * per_worker + t
            base = chunk * _LANE
            pltpu.sync_copy(idx_hbm.at[pl.ds(base, _LANE)], idx_v)
            pltpu.async_copy(m2_hbm.at[idx_v], rows_v, sem).wait()
            pltpu.sync_copy(rows_v, w_hbm.at[0, pl.ds(base, _LANE)])
            pltpu.async_copy(m1_hbm.at[idx_v], rows_v, sem).wait()
            pltpu.sync_copy(rows_v, w_hbm.at[1, pl.ds(base, _LANE)])

    return gather_kernel(m1, m2, idx_flat)


def _prep_kernel(y_ref, v1_ref, v2_ref, m1_ref, m2_ref,
                 a1_ref, a2_ref, u1_ref, u2_ref, sem):
    b = v1_ref.shape[0]
    for i in range(b):
        pltpu.make_async_copy(m1_ref.at[pl.ds(y_ref[i], 1)],
                              a1_ref.at[pl.ds(i, 1)], sem).start()
        pltpu.make_async_copy(m2_ref.at[pl.ds(y_ref[i], 1)],
                              a2_ref.at[pl.ds(i, 1)], sem).start()
    for i in range(2 * b):
        pltpu.make_async_copy(m1_ref.at[pl.ds(0, 1)],
                              a1_ref.at[pl.ds(0, 1)], sem).wait()
    rows1 = a1_ref[...]
    rows2 = a2_ref[...]
    pos1 = rows1 * _MOM + v1_ref[...] * (1.0 - _MOM)
    pos2 = rows2 * _MOM + v2_ref[...] * (1.0 - _MOM)
    n1 = jnp.sqrt(jnp.sum(pos1 * pos1, axis=1, keepdims=True))
    n2 = jnp.sqrt(jnp.sum(pos2 * pos2, axis=1, keepdims=True))
    u1_ref[...] = pos1 / n1
    u2_ref[...] = pos2 / n2


def _prep(y, v1, v2, m1, m2):
    b, d = v1.shape
    return pl.pallas_call(
        _prep_kernel,
        in_specs=[
            pl.BlockSpec(memory_space=_SMEM),
            pl.BlockSpec(memory_space=_VMEM),
            pl.BlockSpec(memory_space=_VMEM),
            pl.BlockSpec(memory_space=_HBM),
            pl.BlockSpec(memory_space=_HBM),
        ],
        out_specs=[pl.BlockSpec(memory_space=_VMEM)] * 4,
        out_shape=[jax.ShapeDtypeStruct((b, d), jnp.float32)] * 4,
        scratch_shapes=[pltpu.SemaphoreType.DMA],
    )(y, v1, v2, m1, m2)


def _block_scales(w1, a):
    g = lax.dot_general(w1, a, (((1,), (1,)), ((), ())),
                        preferred_element_type=jnp.float32,
                        precision=lax.Precision.HIGHEST)
    wn = jnp.sum(w1 * w1, axis=1, keepdims=True)
    an = jnp.sum(a * a, axis=1)
    ssq = wn - 2.0 * g + an[None, :]
    nrm = jnp.sqrt(jnp.maximum(ssq, _SSQ_CLAMP))
    return (_LOG2E / _T) / nrm


def _copy_schedule(n):
    p1_slab = int(n * 0.6) // (32 * 8) * 8
    p1_rows = p1_slab * 32
    rest = n - p1_rows
    p2_slab = rest // (16 * 8) * 8
    p2_tail = rest - p2_slab * 16
    return p1_slab, p1_rows, p2_slab, p2_tail


def _slab_copy(src_ref, dst_ref, sem, step, nsteps, base, slab, tail):
    start = base + step * slab

    @pl.when(step > 0)
    def _():
        pltpu.make_async_copy(src_ref.at[pl.ds(0, slab)],
                              dst_ref.at[pl.ds(0, slab)], sem).wait()

    pltpu.make_async_copy(src_ref.at[pl.ds(start, slab)],
                          dst_ref.at[pl.ds(start, slab)], sem).start()

    @pl.when(step == nsteps - 1)
    def _():
        if tail:
            t0 = base + nsteps * slab
            pltpu.make_async_copy(src_ref.at[pl.ds(t0, tail)],
                                  dst_ref.at[pl.ds(t0, tail)], sem).start()
        pltpu.make_async_copy(src_ref.at[pl.ds(0, slab)],
                              dst_ref.at[pl.ds(0, slab)], sem).wait()
        if tail:
            pltpu.make_async_copy(src_ref.at[pl.ds(0, tail)],
                                  dst_ref.at[pl.ds(0, tail)], sem).wait()


def _pass1_kernel(p1_slab, w_ref, a_ref, m1_ref, m2_ref,
                  p_ref, c1_ref, c2_ref, sem1, sem2):
    bb = pl.program_id(0)
    i = pl.program_id(1)
    step = bb * pl.num_programs(1) + i
    nsteps = pl.num_programs(0) * pl.num_programs(1)
    _slab_copy(m1_ref, c1_ref, sem1, step, nsteps, 0, p1_slab, 0)
    _slab_copy(m2_ref, c2_ref, sem2, step, nsteps, 0, p1_slab, 0)

    b_anch = a_ref.shape[1]
    w1 = w_ref[0, 0] + 1e-6
    a = a_ref[0]
    scales = _block_scales(w1, a)

    acc = jnp.zeros((w1.shape[1],), jnp.float32)
    for j in range(b_anch):
        sj = scales[:, j:j + 1]
        e = jnp.exp2((w1 - a[j, :][None, :]) * sj)
        acc = acc + jnp.sum(e, axis=0)

    @pl.when(i == 0)
    def _():
        p_ref[...] = jnp.zeros_like(p_ref)

    p_ref[...] += jnp.broadcast_to(acc[None, None, :], p_ref.shape)


def _pass1(w4, anchors, m1, m2, p1_slab):
    nb, kk, d = w4.shape[1], w4.shape[2], w4.shape[3]
    b = anchors.shape[1]
    return pl.pallas_call(
        functools.partial(_pass1_kernel, p1_slab),
        grid=(2, nb),
        in_specs=[
            pl.BlockSpec((1, 1, kk, d), lambda bb, i: (bb, i, 0, 0)),
            pl.BlockSpec((1, b, d), lambda bb, i: (bb, 0, 0)),
            pl.BlockSpec(memory_space=_HBM),
            pl.BlockSpec(memory_space=_HBM),
        ],
        out_specs=[
            pl.BlockSpec((1, 8, d), lambda bb, i: (bb, 0, 0)),
            pl.BlockSpec(memory_space=_HBM),
            pl.BlockSpec(memory_space=_HBM),
        ],
        out_shape=[
            jax.ShapeDtypeStruct((2, 8, d), jnp.float32),
            jax.ShapeDtypeStruct(m1.shape, m1.dtype),
            jax.ShapeDtypeStruct(m2.shape, m2.dtype),
        ],
        scratch_shapes=[pltpu.SemaphoreType.DMA, pltpu.SemaphoreType.DMA],
        compiler_params=pltpu.CompilerParams(
            dimension_semantics=("arbitrary", "arbitrary")),
    )(w4, anchors, m1, m2)


def _pass2_kernel(branch, sched, w_ref, a_ref, z_ref, m_ref, cin_ref,
                  o_ref, c_ref, sem):
    _, p1_rows, p2_slab, p2_tail = sched
    step = pl.program_id(0)
    nsteps = pl.num_programs(0)
    _slab_copy(m_ref, c_ref, sem, step, nsteps, p1_rows, p2_slab, p2_tail)

    b_anch = a_ref.shape[1]
    w1 = w_ref[0, 0] + 1e-6
    a = a_ref[0]
    lg_inv_z = z_ref[branch]
    scales = _block_scales(w1, a)

    for j in range(b_anch):
        sj = scales[:, j:j + 1]
        o_ref[pl.ds(j, 1)] = jnp.exp2(
            (w1 - a[j, :][None, :]) * sj + lg_inv_z)[None]


def _pass2(branch, w4, anchors, lg_inv_z, m, c_in, sched):
    nb, kk, d = w4.shape[1], w4.shape[2], w4.shape[3]
    b = anchors.shape[1]
    return pl.pallas_call(
        functools.partial(_pass2_kernel, branch, sched),
        grid=(nb,),
        in_specs=[
            pl.BlockSpec((1, 1, kk, d), lambda i: (branch, i, 0, 0)),
            pl.BlockSpec((1, b, d), lambda i: (branch, 0, 0)),
            pl.BlockSpec(memory_space=_SMEM),
            pl.BlockSpec(memory_space=_HBM),
            pl.BlockSpec(memory_space=_HBM),
        ],
        out_specs=[
            pl.BlockSpec((b, kk, d), lambda i: (i, 0, 0)),
            pl.BlockSpec(memory_space=_HBM),
        ],
        out_shape=[
            jax.ShapeDtypeStruct((nb * b, kk, d), jnp.float32),
            jax.ShapeDtypeStruct(m.shape, m.dtype),
        ],
        input_output_aliases={4: 1},
        scratch_shapes=[pltpu.SemaphoreType.DMA],
        compiler_params=pltpu.CompilerParams(
            dimension_semantics=("arbitrary",)),
    )(w4, anchors, lg_inv_z, m, c_in)


def _patch_kernel(y_ref, u1_ref, u2_ref, c1_ref, c2_ref,
                  nm1_ref, nm2_ref, sem):
    b = u1_ref.shape[0]
    for i in range(b):
        yi = y_ref[i]
        dup = jnp.bool_(False)
        for j in range(i + 1, b):
            dup = jnp.logical_or(dup, y_ref[j] == yi)
        keep = jnp.logical_not(dup)

        @pl.when(keep)
        def _():
            pltpu.make_async_copy(u1_ref.at[pl.ds(i, 1)],
                                  nm1_ref.at[pl.ds(yi, 1)], sem).start()
            pltpu.make_async_copy(u2_ref.at[pl.ds(i, 1)],
                                  nm2_ref.at[pl.ds(yi, 1)], sem).start()

        @pl.when(keep)
        def _():
            pltpu.make_async_copy(u1_ref.at[pl.ds(i, 1)],
                                  nm1_ref.at[pl.ds(yi, 1)], sem).wait()
            pltpu.make_async_copy(u2_ref.at[pl.ds(i, 1)],
                                  nm2_ref.at[pl.ds(yi, 1)], sem).wait()


def _patch(y, u1, u2, c1, c2):
    return pl.pallas_call(
        _patch_kernel,
        in_specs=[
            pl.BlockSpec(memory_space=_SMEM),
            pl.BlockSpec(memory_space=_VMEM),
            pl.BlockSpec(memory_space=_VMEM),
            pl.BlockSpec(memory_space=_HBM),
            pl.BlockSpec(memory_space=_HBM),
        ],
        out_specs=[pl.BlockSpec(memory_space=_HBM)] * 2,
        out_shape=[jax.ShapeDtypeStruct(c1.shape, c1.dtype)] * 2,
        input_output_aliases={3: 0, 4: 1},
        scratch_shapes=[pltpu.SemaphoreType.DMA],
    )(y, u1, u2, c1, c2)


def kernel(v1, v2, y, idx, memory_v1, memory_v2):
    b, d = v1.shape
    n = memory_v1.shape[0]
    kk = idx.shape[1]
    total = b * kk
    sched = _copy_schedule(n)

    w = _sc_gather(memory_v1, memory_v2, idx.reshape(total), total)
    w4 = w.reshape(2, b, kk, d)

    a1, a2, u1, u2 = _prep(y, v1, v2, memory_v1, memory_v2)
    anchors = jnp.stack([a2, a1])

    part, c1, c2 = _pass1(w4, anchors, memory_v1, memory_v2, sched[0])
    s = jnp.sum(part[:, 0, :], axis=1)
    z = s / jnp.float32(b * b * kk * d) * jnp.float32(n)
    lg_inv_z = (-jnp.log2(z)).astype(jnp.float32)

    out_v1, c1 = _pass2(0, w4, anchors, lg_inv_z, memory_v1, c1, sched)
    out_v2, c2 = _pass2(1, w4, anchors, lg_inv_z, memory_v2, c2, sched)
    new_m1, new_m2 = _patch(y, u1, u2, c1, c2)
    return (out_v1, out_v2, new_m1, new_m2)


# speedup vs baseline: 16.2008x; 16.2008x over previous
import functools

import jax
import jax.numpy as jnp
from jax import lax
from jax.experimental import pallas as pl
from jax.experimental.pallas import tpu as pltpu
from jax.experimental.pallas import tpu_sc as plsc

_T = 0.05
_MOM = 0.5
_LOG2E = 1.4426950408889634
_SSQ_CLAMP = 1e-6

_NC = 2
_NS = 16
_NW = _NC * _NS
_LANE = 128

_HBM = pltpu.MemorySpace.HBM
_VMEM = pltpu.MemorySpace.VMEM
_SMEM = pltpu.MemorySpace.SMEM


def _sc_gather(m1, m2, idx_flat, total):
    d = m1.shape[1]
    n_chunks = total // _LANE
    per_worker = n_chunks // _NW
    mesh = plsc.VectorSubcoreMesh(
        core_axis_name="c", subcore_axis_name="s",
        num_cores=_NC, num_subcores=_NS)

    @functools.partial(
        pl.kernel,
        out_type=jax.ShapeDtypeStruct((2, total, d), jnp.float32),
        mesh=mesh,
        scratch_types=[
            pltpu.VMEM((_LANE,), jnp.int32),
            pltpu.VMEM((_LANE, d), jnp.float32),
            pltpu.SemaphoreType.DMA,
        ],
    )
    def gather_kernel(m1_hbm, m2_hbm, idx_hbm, w_hbm, idx_v, rows_v, sem):
        wid = lax.axis_index("s") * _NC + lax.axis_index("c")
        for t in range(per_worker):
            chunk = wid * per_worker + t
            base = chunk * _LANE
            pltpu.sync_copy(idx_hbm.at[pl.ds(base, _LANE)], idx_v)
            pltpu.async_copy(m2_hbm.at[idx_v], rows_v, sem).wait()
            pltpu.sync_copy(rows_v, w_hbm.at[0, pl.ds(base, _LANE)])
            pltpu.async_copy(m1_hbm.at[idx_v], rows_v, sem).wait()
            pltpu.sync_copy(rows_v, w_hbm.at[1, pl.ds(base, _LANE)])

    return gather_kernel(m1, m2, idx_flat)


def _prep_kernel(y_ref, v1_ref, v2_ref, m1_ref, m2_ref,
                 a1_ref, a2_ref, u1_ref, u2_ref, sem):
    b = v1_ref.shape[0]
    for i in range(b):
        pltpu.make_async_copy(m1_ref.at[pl.ds(y_ref[i], 1)],
                              a1_ref.at[pl.ds(i, 1)], sem).start()
        pltpu.make_async_copy(m2_ref.at[pl.ds(y_ref[i], 1)],
                              a2_ref.at[pl.ds(i, 1)], sem).start()
    for i in range(2 * b):
        pltpu.make_async_copy(m1_ref.at[pl.ds(0, 1)],
                              a1_ref.at[pl.ds(0, 1)], sem).wait()
    rows1 = a1_ref[...]
    rows2 = a2_ref[...]
    pos1 = rows1 * _MOM + v1_ref[...] * (1.0 - _MOM)
    pos2 = rows2 * _MOM + v2_ref[...] * (1.0 - _MOM)
    n1 = jnp.sqrt(jnp.sum(pos1 * pos1, axis=1, keepdims=True))
    n2 = jnp.sqrt(jnp.sum(pos2 * pos2, axis=1, keepdims=True))
    u1_ref[...] = pos1 / n1
    u2_ref[...] = pos2 / n2


def _prep(y, v1, v2, m1, m2):
    b, d = v1.shape
    return pl.pallas_call(
        _prep_kernel,
        in_specs=[
            pl.BlockSpec(memory_space=_SMEM),
            pl.BlockSpec(memory_space=_VMEM),
            pl.BlockSpec(memory_space=_VMEM),
            pl.BlockSpec(memory_space=_HBM),
            pl.BlockSpec(memory_space=_HBM),
        ],
        out_specs=[pl.BlockSpec(memory_space=_VMEM)] * 4,
        out_shape=[jax.ShapeDtypeStruct((b, d), jnp.float32)] * 4,
        scratch_shapes=[pltpu.SemaphoreType.DMA],
    )(y, v1, v2, m1, m2)


def _block_scales(w1, a):
    g = lax.dot_general(w1, a, (((1,), (1,)), ((), ())),
                        preferred_element_type=jnp.float32,
                        precision=lax.Precision.HIGHEST)
    wn = jnp.sum(w1 * w1, axis=1, keepdims=True)
    an = jnp.sum(a * a, axis=1)
    ssq = wn - 2.0 * g + an[None, :]
    nrm = jnp.sqrt(jnp.maximum(ssq, _SSQ_CLAMP))
    return (_LOG2E / _T) / nrm


def _pass1_kernel(w_ref, a_ref, m1_ref, p_ref, c1_ref):
    i = pl.program_id(1)
    c1_ref[...] = m1_ref[...]

    b_anch = a_ref.shape[1]
    w1 = w_ref[0, 0] + 1e-6
    a = a_ref[0]
    scales = _block_scales(w1, a)

    acc = jnp.zeros((w1.shape[1],), jnp.float32)
    for j in range(b_anch):
        sj = scales[:, j:j + 1]
        e = jnp.exp2((w1 - a[j, :][None, :]) * sj)
        acc = acc + jnp.sum(e, axis=0)

    @pl.when(i == 0)
    def _():
        p_ref[...] = jnp.zeros_like(p_ref)

    p_ref[...] += jnp.broadcast_to(acc[None, None, :], p_ref.shape)


def _pass1(w4, anchors, m1):
    nb, kk, d = w4.shape[1], w4.shape[2], w4.shape[3]
    b = anchors.shape[1]
    n = m1.shape[0]
    rows = -(-n // (2 * nb)) // 8 * 8 + 8
    return pl.pallas_call(
        _pass1_kernel,
        grid=(2, nb),
        in_specs=[
            pl.BlockSpec((1, 1, kk, d), lambda bb, i: (bb, i, 0, 0)),
            pl.BlockSpec((1, b, d), lambda bb, i: (bb, 0, 0)),
            pl.BlockSpec((rows, d), lambda bb, i: (bb * 16 + i, 0)),
        ],
        out_specs=[
            pl.BlockSpec((1, 8, d), lambda bb, i: (bb, 0, 0)),
            pl.BlockSpec((rows, d), lambda bb, i: (bb * 16 + i, 0)),
        ],
        out_shape=[
            jax.ShapeDtypeStruct((2, 8, d), jnp.float32),
            jax.ShapeDtypeStruct(m1.shape, m1.dtype),
        ],
        compiler_params=pltpu.CompilerParams(
            dimension_semantics=("arbitrary", "arbitrary")),
    )(w4, anchors, m1)


def _pass2_copy_kernel(branch, w_ref, a_ref, z_ref, m_ref, o_ref, c_ref):
    c_ref[...] = m_ref[...]
    _pass2_body(branch, w_ref, a_ref, z_ref, o_ref)


def _pass2_plain_kernel(branch, w_ref, a_ref, z_ref, o_ref):
    _pass2_body(branch, w_ref, a_ref, z_ref, o_ref)


def _pass2_body(branch, w_ref, a_ref, z_ref, o_ref):
    b_anch = a_ref.shape[1]
    w1 = w_ref[0, 0] + 1e-6
    a = a_ref[0]
    lg_inv_z = z_ref[branch]
    scales = _block_scales(w1, a)

    for j in range(b_anch):
        sj = scales[:, j:j + 1]
        o_ref[pl.ds(j, 1)] = jnp.exp2(
            (w1 - a[j, :][None, :]) * sj + lg_inv_z)[None]


def _pass2(branch, w4, anchors, lg_inv_z, m=None):
    nb, kk, d = w4.shape[1], w4.shape[2], w4.shape[3]
    b = anchors.shape[1]
    in_specs = [
        pl.BlockSpec((1, 1, kk, d), lambda i: (branch, i, 0, 0)),
        pl.BlockSpec((1, b, d), lambda i: (branch, 0, 0)),
        pl.BlockSpec(memory_space=_SMEM),
    ]
    out_specs = [pl.BlockSpec((b, kk, d), lambda i: (i, 0, 0))]
    out_shape = [jax.ShapeDtypeStruct((nb * b, kk, d), jnp.float32)]
    if m is not None:
        rows = -(-m.shape[0] // nb) // 8 * 8 + 8
        in_specs.append(pl.BlockSpec((rows, d), lambda i: (i, 0)))
        out_specs.append(pl.BlockSpec((rows, d), lambda i: (i, 0)))
        out_shape.append(jax.ShapeDtypeStruct(m.shape, m.dtype))
        body = functools.partial(_pass2_copy_kernel, branch)
        args = (w4, anchors, lg_inv_z, m)
    else:
        body = functools.partial(_pass2_plain_kernel, branch)
        args = (w4, anchors, lg_inv_z)
    res = pl.pallas_call(
        body,
        grid=(nb,),
        in_specs=in_specs,
        out_specs=out_specs,
        out_shape=out_shape,
        compiler_params=pltpu.CompilerParams(
            dimension_semantics=("arbitrary",)),
    )(*args)
    return res[0] if m is None else (res[0], res[1])


def _patch_kernel(y_ref, u1_ref, u2_ref, c1_ref, c2_ref,
                  nm1_ref, nm2_ref, sem):
    b = u1_ref.shape[0]
    for i in range(b):
        yi = y_ref[i]
        dup = jnp.bool_(False)
        for j in range(i + 1, b):
            dup = jnp.logical_or(dup, y_ref[j] == yi)
        keep = jnp.logical_not(dup)

        @pl.when(keep)
        def _():
            pltpu.make_async_copy(u1_ref.at[pl.ds(i, 1)],
                                  nm1_ref.at[pl.ds(yi, 1)], sem).start()
            pltpu.make_async_copy(u2_ref.at[pl.ds(i, 1)],
                                  nm2_ref.at[pl.ds(yi, 1)], sem).start()

        @pl.when(keep)
        def _():
            pltpu.make_async_copy(u1_ref.at[pl.ds(i, 1)],
                                  nm1_ref.at[pl.ds(yi, 1)], sem).wait()
            pltpu.make_async_copy(u2_ref.at[pl.ds(i, 1)],
                                  nm2_ref.at[pl.ds(yi, 1)], sem).wait()


def _patch(y, u1, u2, c1, c2):
    return pl.pallas_call(
        _patch_kernel,
        in_specs=[
            pl.BlockSpec(memory_space=_SMEM),
            pl.BlockSpec(memory_space=_VMEM),
            pl.BlockSpec(memory_space=_VMEM),
            pl.BlockSpec(memory_space=_HBM),
            pl.BlockSpec(memory_space=_HBM),
        ],
        out_specs=[pl.BlockSpec(memory_space=_HBM)] * 2,
        out_shape=[jax.ShapeDtypeStruct(c1.shape, c1.dtype)] * 2,
        input_output_aliases={3: 0, 4: 1},
        scratch_shapes=[pltpu.SemaphoreType.DMA],
    )(y, u1, u2, c1, c2)


def kernel(v1, v2, y, idx, memory_v1, memory_v2):
    b, d = v1.shape
    n = memory_v1.shape[0]
    kk = idx.shape[1]
    total = b * kk

    w = _sc_gather(memory_v1, memory_v2, idx.reshape(total), total)
    w4 = w.reshape(2, b, kk, d)

    a1, a2, u1, u2 = _prep(y, v1, v2, memory_v1, memory_v2)
    anchors = jnp.stack([a2, a1])

    part, c1 = _pass1(w4, anchors, memory_v1)
    s = jnp.sum(part[:, 0, :], axis=1)
    z = s / jnp.float32(b * b * kk * d) * jnp.float32(n)
    lg_inv_z = (-jnp.log2(z)).astype(jnp.float32)

    out_v1 = _pass2(0, w4, anchors, lg_inv_z)
    out_v2, c2 = _pass2(1, w4, anchors, lg_inv_z, memory_v2)
    new_m1, new_m2 = _patch(y, u1, u2, c1, c2)
    return (out_v1, out_v2, new_m1, new_m2)


# speedup vs baseline: 16.4170x; 1.0133x over previous
import functools

import jax
import jax.numpy as jnp
from jax import lax
from jax.experimental import pallas as pl
from jax.experimental.pallas import tpu as pltpu
from jax.experimental.pallas import tpu_sc as plsc

_T = 0.05
_MOM = 0.5
_LOG2E = 1.4426950408889634
_SSQ_CLAMP = 1e-6

_NC = 2
_NS = 16
_NW = _NC * _NS
_LANE = 128

_P1_B2_BLK = 400

_HBM = pltpu.MemorySpace.HBM
_VMEM = pltpu.MemorySpace.VMEM
_SMEM = pltpu.MemorySpace.SMEM


def _sc_gather(m1, m2, idx_flat, total):
    d = m1.shape[1]
    n_chunks = total // _LANE
    per_worker = n_chunks // _NW
    mesh = plsc.VectorSubcoreMesh(
        core_axis_name="c", subcore_axis_name="s",
        num_cores=_NC, num_subcores=_NS)

    @functools.partial(
        pl.kernel,
        out_type=jax.ShapeDtypeStruct((2, total, d), jnp.float32),
        mesh=mesh,
        scratch_types=[
            pltpu.VMEM((_LANE,), jnp.int32),
            pltpu.VMEM((_LANE, d), jnp.float32),
            pltpu.SemaphoreType.DMA,
        ],
    )
    def gather_kernel(m1_hbm, m2_hbm, idx_hbm, w_hbm, idx_v, rows_v, sem):
        wid = lax.axis_index("s") * _NC + lax.axis_index("c")
        for t in range(per_worker):
            chunk = wid * per_worker + t
            base = chunk * _LANE
            pltpu.sync_copy(idx_hbm.at[pl.ds(base, _LANE)], idx_v)
            pltpu.async_copy(m2_hbm.at[idx_v], rows_v, sem).wait()
            pltpu.sync_copy(rows_v, w_hbm.at[0, pl.ds(base, _LANE)])
            pltpu.async_copy(m1_hbm.at[idx_v], rows_v, sem).wait()
            pltpu.sync_copy(rows_v, w_hbm.at[1, pl.ds(base, _LANE)])

    return gather_kernel(m1, m2, idx_flat)


def _block_scales(w1, a):
    g = lax.dot_general(w1, a, (((1,), (1,)), ((), ())),
                        preferred_element_type=jnp.float32,
                        precision=lax.Precision.HIGHEST)
    wn = jnp.sum(w1 * w1, axis=1, keepdims=True)
    an = jnp.sum(a * a, axis=1)
    ssq = wn - 2.0 * g + an[None, :]
    nrm = jnp.sqrt(jnp.maximum(ssq, _SSQ_CLAMP))
    return (_LOG2E / _T) / nrm


def _anchors_of(a_ref, bb):
    a0 = a_ref[0]
    a1 = a_ref[1]
    return jnp.where(bb == 0, a0, a1)


def _pass1_kernel(w_ref, y_ref, v1_ref, v2_ref, m1r_ref, m2r_ref,
                  m1_ref, m2t_ref,
                  p_ref, c1_ref, c2t_ref, a_ref, u1_ref, u2_ref, sem):
    bb = pl.program_id(0)
    i = pl.program_id(1)
    step = bb * pl.num_programs(1) + i

    c1_ref[...] = m1_ref[...]
    c2t_ref[...] = m2t_ref[...]

    @pl.when(step == 0)
    def _():
        b = v1_ref.shape[0]
        for t in range(b):
            pltpu.make_async_copy(m1r_ref.at[pl.ds(y_ref[t], 1)],
                                  u1_ref.at[pl.ds(t, 1)], sem).start()
            pltpu.make_async_copy(m2r_ref.at[pl.ds(y_ref[t], 1)],
                                  u2_ref.at[pl.ds(t, 1)], sem).start()
        for t in range(2 * b):
            pltpu.make_async_copy(m1r_ref.at[pl.ds(0, 1)],
                                  u1_ref.at[pl.ds(0, 1)], sem).wait()
        rows1 = u1_ref[...]
        rows2 = u2_ref[...]
        a_ref[0] = rows2
        a_ref[1] = rows1
        pos1 = rows1 * _MOM + v1_ref[...] * (1.0 - _MOM)
        pos2 = rows2 * _MOM + v2_ref[...] * (1.0 - _MOM)
        n1 = jnp.sqrt(jnp.sum(pos1 * pos1, axis=1, keepdims=True))
        n2 = jnp.sqrt(jnp.sum(pos2 * pos2, axis=1, keepdims=True))
        u1_ref[...] = pos1 / n1
        u2_ref[...] = pos2 / n2

    b_anch = a_ref.shape[1]
    w1 = w_ref[0, 0] + 1e-6
    a = _anchors_of(a_ref, bb)
    scales = _block_scales(w1, a)

    acc = jnp.zeros((w1.shape[1],), jnp.float32)
    for j in range(b_anch):
        sj = scales[:, j:j + 1]
        e = jnp.exp2((w1 - a[j, :][None, :]) * sj)
        acc = acc + jnp.sum(e, axis=0)

    @pl.when(i == 0)
    def _():
        p_ref[...] = jnp.zeros_like(p_ref)

    p_ref[...] += jnp.broadcast_to(acc[None, None, :], p_ref.shape)


def _pass1(w4, y, v1, v2, m1, m2):
    nb, kk, d = w4.shape[1], w4.shape[2], w4.shape[3]
    b = v1.shape[0]
    n = m1.shape[0]
    rows1 = -(-n // (2 * nb)) // 8 * 8 + 8
    t_base = n // _P1_B2_BLK - 2 * nb
    return pl.pallas_call(
        _pass1_kernel,
        grid=(2, nb),
        in_specs=[
            pl.BlockSpec((1, 1, kk, d), lambda bb, i: (bb, i, 0, 0)),
            pl.BlockSpec(memory_space=_SMEM),
            pl.BlockSpec(memory_space=_VMEM),
            pl.BlockSpec(memory_space=_VMEM),
            pl.BlockSpec(memory_space=_HBM),
            pl.BlockSpec(memory_space=_HBM),
            pl.BlockSpec((rows1, d), lambda bb, i: (bb * 16 + i, 0)),
            pl.BlockSpec((_P1_B2_BLK, d),
                         lambda bb, i: (t_base + bb * 16 + i, 0)),
        ],
        out_specs=[
            pl.BlockSpec((1, 8, d), lambda bb, i: (bb, 0, 0)),
            pl.BlockSpec((rows1, d), lambda bb, i: (bb * 16 + i, 0)),
            pl.BlockSpec((_P1_B2_BLK, d),
                         lambda bb, i: (t_base + bb * 16 + i, 0)),
            pl.BlockSpec((2, b, d), lambda bb, i: (0, 0, 0)),
            pl.BlockSpec((b, d), lambda bb, i: (0, 0)),
            pl.BlockSpec((b, d), lambda bb, i: (0, 0)),
        ],
        out_shape=[
            jax.ShapeDtypeStruct((2, 8, d), jnp.float32),
            jax.ShapeDtypeStruct(m1.shape, m1.dtype),
            jax.ShapeDtypeStruct(m2.shape, m2.dtype),
            jax.ShapeDtypeStruct((2, b, d), jnp.float32),
            jax.ShapeDtypeStruct((b, d), jnp.float32),
            jax.ShapeDtypeStruct((b, d), jnp.float32),
        ],
        scratch_shapes=[pltpu.SemaphoreType.DMA],
        compiler_params=pltpu.CompilerParams(
            dimension_semantics=("arbitrary", "arbitrary")),
    )(w4, y, v1, v2, m1, m2, m1, m2)


def _pass2_kernel(branch, w_ref, a_ref, z_ref, m2_ref, c2in_ref,
                  o_ref, c2_ref):
    c2_ref[...] = m2_ref[...]

    b_anch = a_ref.shape[1]
    w1 = w_ref[0, 0] + 1e-6
    a = a_ref[branch]
    lg_inv_z = z_ref[branch]
    scales = _block_scales(w1, a)

    for j in range(b_anch):
        sj = scales[:, j:j + 1]
        o_ref[pl.ds(j, 1)] = jnp.exp2(
            (w1 - a[j, :][None, :]) * sj + lg_inv_z)[None]


def _pass2(branch, w4, anchors, lg_inv_z, m2, c2_in):
    nb, kk, d = w4.shape[1], w4.shape[2], w4.shape[3]
    b = anchors.shape[1]
    n = m2.shape[0]
    head = n - 2 * nb * _P1_B2_BLK
    rows2 = -(-head // (2 * nb)) // 8 * 8 + 8
    off = branch * nb
    out = pl.pallas_call(
        functools.partial(_pass2_kernel, branch),
        grid=(nb,),
        in_specs=[
            pl.BlockSpec((1, 1, kk, d), lambda i: (branch, i, 0, 0)),
            pl.BlockSpec((2, b, d), lambda i: (0, 0, 0)),
            pl.BlockSpec(memory_space=_SMEM),
            pl.BlockSpec((rows2, d), lambda i: (off + i, 0)),
            pl.BlockSpec(memory_space=_HBM),
        ],
        out_specs=[
            pl.BlockSpec((b, kk, d), lambda i: (i, 0, 0)),
            pl.BlockSpec((rows2, d), lambda i: (off + i, 0)),
        ],
        out_shape=[
            jax.ShapeDtypeStruct((nb * b, kk, d), jnp.float32),
            jax.ShapeDtypeStruct(m2.shape, m2.dtype),
        ],
        input_output_aliases={4: 1},
        compiler_params=pltpu.CompilerParams(
            dimension_semantics=("arbitrary",)),
    )(w4, anchors, lg_inv_z, m2, c2_in)
    return out


def _patch_kernel(y_ref, u1_ref, u2_ref, c1_ref, c2_ref,
                  nm1_ref, nm2_ref, sem):
    b = u1_ref.shape[0]
    for i in range(b):
        yi = y_ref[i]
        dup = jnp.bool_(False)
        for j in range(i + 1, b):
            dup = jnp.logical_or(dup, y_ref[j] == yi)
        keep = jnp.logical_not(dup)

        @pl.when(keep)
        def _():
            pltpu.make_async_copy(u1_ref.at[pl.ds(i, 1)],
                                  nm1_ref.at[pl.ds(yi, 1)], sem).start()
            pltpu.make_async_copy(u2_ref.at[pl.ds(i, 1)],
                                  nm2_ref.at[pl.ds(yi, 1)], sem).start()

        @pl.when(keep)
        def _():
            pltpu.make_async_copy(u1_ref.at[pl.ds(i, 1)],
                                  nm1_ref.at[pl.ds(yi, 1)], sem).wait()
            pltpu.make_async_copy(u2_ref.at[pl.ds(i, 1)],
                                  nm2_ref.at[pl.ds(yi, 1)], sem).wait()


def _patch(y, u1, u2, c1, c2):
    return pl.pallas_call(
        _patch_kernel,
        in_specs=[
            pl.BlockSpec(memory_space=_SMEM),
            pl.BlockSpec(memory_space=_VMEM),
            pl.BlockSpec(memory_space=_VMEM),
            pl.BlockSpec(memory_space=_HBM),
            pl.BlockSpec(memory_space=_HBM),
        ],
        out_specs=[pl.BlockSpec(memory_space=_HBM)] * 2,
        out_shape=[jax.ShapeDtypeStruct(c1.shape, c1.dtype)] * 2,
        input_output_aliases={3: 0, 4: 1},
        scratch_shapes=[pltpu.SemaphoreType.DMA],
    )(y, u1, u2, c1, c2)


def kernel(v1, v2, y, idx, memory_v1, memory_v2):
    b, d = v1.shape
    n = memory_v1.shape[0]
    kk = idx.shape[1]
    total = b * kk

    w = _sc_gather(memory_v1, memory_v2, idx.reshape(total), total)
    w4 = w.reshape(2, b, kk, d)

    part, c1, c2t, anchors, u1, u2 = _pass1(w4, y, v1, v2,
                                            memory_v1, memory_v2)
    s = jnp.sum(part[:, 0, :], axis=1)
    z = s / jnp.float32(b * b * kk * d) * jnp.float32(n)
    lg_inv_z = (-jnp.log2(z)).astype(jnp.float32)

    out_v1, c2h = _pass2(0, w4, anchors, lg_inv_z, memory_v2, c2t)
    out_v2, c2 = _pass2(1, w4, anchors, lg_inv_z, memory_v2, c2h)
    new_m1, new_m2 = _patch(y, u1, u2, c1, c2)
    return (out_v1, out_v2, new_m1, new_m2)


# speedup vs baseline: 16.7845x; 1.0224x over previous
import functools

import jax
import jax.numpy as jnp
from jax import lax
from jax.experimental import pallas as pl
from jax.experimental.pallas import tpu as pltpu
from jax.experimental.pallas import tpu_sc as plsc

_T = 0.05
_MOM = 0.5
_LOG2E = 1.4426950408889634
_SSQ_CLAMP = 1e-6

_NC = 2
_NS = 16
_NW = _NC * _NS
_LANE = 128

_P1_B2_BLK = 1000

_HBM = pltpu.MemorySpace.HBM
_VMEM = pltpu.MemorySpace.VMEM
_SMEM = pltpu.MemorySpace.SMEM


def _sc_gather(m1, m2, idx_flat, total):
    d = m1.shape[1]
    n_chunks = total // _LANE
    per_worker = n_chunks // _NW
    mesh = plsc.VectorSubcoreMesh(
        core_axis_name="c", subcore_axis_name="s",
        num_cores=_NC, num_subcores=_NS)

    @functools.partial(
        pl.kernel,
        out_type=jax.ShapeDtypeStruct((2, total, d), jnp.float32),
        mesh=mesh,
        scratch_types=[
            pltpu.VMEM((2, _LANE), jnp.int32),
            pltpu.VMEM((4, _LANE, d), jnp.float32),
            pltpu.SemaphoreType.DMA,
            pltpu.SemaphoreType.DMA,
        ],
    )
    def gather_kernel(m1_hbm, m2_hbm, idx_hbm, w_hbm,
                      idx_v, rows_v, gsem, osem):
        wid = lax.axis_index("s") * _NC + lax.axis_index("c")
        for t in range(per_worker):
            base = (wid * per_worker + t) * _LANE
            pltpu.sync_copy(idx_hbm.at[pl.ds(base, _LANE)], idx_v.at[t])
        gathers = []
        for t in range(per_worker):
            gathers.append(pltpu.async_copy(
                m2_hbm.at[idx_v.at[t]], rows_v.at[2 * t], gsem))
            gathers.append(pltpu.async_copy(
                m1_hbm.at[idx_v.at[t]], rows_v.at[2 * t + 1], gsem))
        for h in gathers:
            h.wait()
        outs = []
        for t in range(per_worker):
            base = (wid * per_worker + t) * _LANE
            outs.append(pltpu.async_copy(
                rows_v.at[2 * t], w_hbm.at[0, pl.ds(base, _LANE)], osem))
            outs.append(pltpu.async_copy(
                rows_v.at[2 * t + 1], w_hbm.at[1, pl.ds(base, _LANE)], osem))
        for h in outs:
            h.wait()

    return gather_kernel(m1, m2, idx_flat)


def _block_scales(w1, a):
    g = lax.dot_general(w1, a, (((1,), (1,)), ((), ())),
                        preferred_element_type=jnp.float32,
                        precision=lax.Precision.HIGHEST)
    wn = jnp.sum(w1 * w1, axis=1, keepdims=True)
    an = jnp.sum(a * a, axis=1)
    ssq = wn - 2.0 * g + an[None, :]
    nrm = jnp.sqrt(jnp.maximum(ssq, _SSQ_CLAMP))
    return (_LOG2E / _T) / nrm


def _anchors_of(a_ref, bb):
    a0 = a_ref[0]
    a1 = a_ref[1]
    return jnp.where(bb == 0, a0, a1)


def _pass1_kernel(w_ref, y_ref, v1_ref, v2_ref, m1r_ref, m2r_ref,
                  m1_ref, m2t_ref,
                  p_ref, c1_ref, c2t_ref, a_ref, u1_ref, u2_ref, sem):
    bb = pl.program_id(0)
    i = pl.program_id(1)
    step = bb * pl.num_programs(1) + i

    c1_ref[...] = m1_ref[...]
    c2t_ref[...] = m2t_ref[...]

    @pl.when(step == 0)
    def _():
        b = v1_ref.shape[0]
        for t in range(b):
            pltpu.make_async_copy(m1r_ref.at[pl.ds(y_ref[t], 1)],
                                  u1_ref.at[pl.ds(t, 1)], sem).start()
            pltpu.make_async_copy(m2r_ref.at[pl.ds(y_ref[t], 1)],
                                  u2_ref.at[pl.ds(t, 1)], sem).start()
        for t in range(2 * b):
            pltpu.make_async_copy(m1r_ref.at[pl.ds(0, 1)],
                                  u1_ref.at[pl.ds(0, 1)], sem).wait()
        rows1 = u1_ref[...]
        rows2 = u2_ref[...]
        a_ref[0] = rows2
        a_ref[1] = rows1
        pos1 = rows1 * _MOM + v1_ref[...] * (1.0 - _MOM)
        pos2 = rows2 * _MOM + v2_ref[...] * (1.0 - _MOM)
        n1 = jnp.sqrt(jnp.sum(pos1 * pos1, axis=1, keepdims=True))
        n2 = jnp.sqrt(jnp.sum(pos2 * pos2, axis=1, keepdims=True))
        u1_ref[...] = pos1 / n1
        u2_ref[...] = pos2 / n2

    b_anch = a_ref.shape[1]
    w1 = w_ref[0, 0] + 1e-6
    a = _anchors_of(a_ref, bb)
    scales = _block_scales(w1, a)

    acc = jnp.zeros((w1.shape[1],), jnp.float32)
    for j in range(b_anch):
        sj = scales[:, j:j + 1]
        e = jnp.exp2((w1 - a[j, :][None, :]) * sj)
        acc = acc + jnp.sum(e, axis=0)

    @pl.when(i == 0)
    def _():
        p_ref[...] = jnp.zeros_like(p_ref)

    p_ref[...] += jnp.broadcast_to(acc[None, None, :], p_ref.shape)


def _pass1(w4, y, v1, v2, m1, m2):
    nb, kk, d = w4.shape[1], w4.shape[2], w4.shape[3]
    b = v1.shape[0]
    n = m1.shape[0]
    rows1 = -(-n // (2 * nb)) // 8 * 8 + 8
    t_base = n // _P1_B2_BLK - 2 * nb
    return pl.pallas_call(
        _pass1_kernel,
        grid=(2, nb),
        in_specs=[
            pl.BlockSpec((1, 1, kk, d), lambda bb, i: (bb, i, 0, 0)),
            pl.BlockSpec(memory_space=_SMEM),
            pl.BlockSpec(memory_space=_VMEM),
            pl.BlockSpec(memory_space=_VMEM),
            pl.BlockSpec(memory_space=_HBM),
            pl.BlockSpec(memory_space=_HBM),
            pl.BlockSpec((rows1, d), lambda bb, i: (bb * 16 + i, 0)),
            pl.BlockSpec((_P1_B2_BLK, d),
                         lambda bb, i: (t_base + bb * 16 + i, 0)),
        ],
        out_specs=[
            pl.BlockSpec((1, 8, d), lambda bb, i: (bb, 0, 0)),
            pl.BlockSpec((rows1, d), lambda bb, i: (bb * 16 + i, 0)),
            pl.BlockSpec((_P1_B2_BLK, d),
                         lambda bb, i: (t_base + bb * 16 + i, 0)),
            pl.BlockSpec((2, b, d), lambda bb, i: (0, 0, 0)),
            pl.BlockSpec((b, d), lambda bb, i: (0, 0)),
            pl.BlockSpec((b, d), lambda bb, i: (0, 0)),
        ],
        out_shape=[
            jax.ShapeDtypeStruct((2, 8, d), jnp.float32),
            jax.ShapeDtypeStruct(m1.shape, m1.dtype),
            jax.ShapeDtypeStruct(m2.shape, m2.dtype),
            jax.ShapeDtypeStruct((2, b, d), jnp.float32),
            jax.ShapeDtypeStruct((b, d), jnp.float32),
            jax.ShapeDtypeStruct((b, d), jnp.float32),
        ],
        scratch_shapes=[pltpu.SemaphoreType.DMA],
        compiler_params=pltpu.CompilerParams(
            dimension_semantics=("arbitrary", "arbitrary")),
    )(w4, y, v1, v2, m1, m2, m1, m2)


def _pass2_kernel(branch, zmul, w_ref, a_ref, p_ref, m2_ref, c2in_ref,
                  o_ref, c2_ref):
    c2_ref[...] = m2_ref[...]

    b_anch = a_ref.shape[1]
    w1 = w_ref[0, 0] + 1e-6
    a = a_ref[branch]
    srow = p_ref[branch, 0:1, :]
    zrow = jnp.sum(srow, axis=1, keepdims=True) * zmul
    lg_inv_z = -jnp.log2(zrow)
    scales = _block_scales(w1, a)

    for j in range(b_anch):
        sj = scales[:, j:j + 1]
        o_ref[pl.ds(j, 1)] = jnp.exp2(
            (w1 - a[j, :][None, :]) * sj + lg_inv_z)[None]


def _pass2(branch, w4, anchors, part, zmul, m2, c2_in):
    nb, kk, d = w4.shape[1], w4.shape[2], w4.shape[3]
    b = anchors.shape[1]
    n = m2.shape[0]
    head = n - 2 * nb * _P1_B2_BLK
    rows2 = -(-head // (2 * nb)) // 8 * 8 + 8
    off = branch * nb
    out = pl.pallas_call(
        functools.partial(_pass2_kernel, branch, zmul),
        grid=(nb,),
        in_specs=[
            pl.BlockSpec((1, 1, kk, d), lambda i: (branch, i, 0, 0)),
            pl.BlockSpec((2, b, d), lambda i: (0, 0, 0)),
            pl.BlockSpec((2, 8, d), lambda i: (0, 0, 0)),
            pl.BlockSpec((rows2, d), lambda i: (off + i, 0)),
            pl.BlockSpec(memory_space=_HBM),
        ],
        out_specs=[
            pl.BlockSpec((b, kk, d), lambda i: (i, 0, 0)),
            pl.BlockSpec((rows2, d), lambda i: (off + i, 0)),
        ],
        out_shape=[
            jax.ShapeDtypeStruct((nb * b, kk, d), jnp.float32),
            jax.ShapeDtypeStruct(m2.shape, m2.dtype),
        ],
        input_output_aliases={4: 1},
        compiler_params=pltpu.CompilerParams(
            dimension_semantics=("arbitrary",)),
    )(w4, anchors, part, m2, c2_in)
    return out


def _patch_kernel(y_ref, u1_ref, u2_ref, c1_ref, c2_ref,
                  nm1_ref, nm2_ref, sem):
    b = u1_ref.shape[0]
    for i in range(b):
        yi = y_ref[i]
        dup = jnp.bool_(False)
        for j in range(i + 1, b):
            dup = jnp.logical_or(dup, y_ref[j] == yi)
        keep = jnp.logical_not(dup)

        @pl.when(keep)
        def _():
            pltpu.make_async_copy(u1_ref.at[pl.ds(i, 1)],
                                  nm1_ref.at[pl.ds(yi, 1)], sem).start()
            pltpu.make_async_copy(u2_ref.at[pl.ds(i, 1)],
                                  nm2_ref.at[pl.ds(yi, 1)], sem).start()

        @pl.when(keep)
        def _():
            pltpu.make_async_copy(u1_ref.at[pl.ds(i, 1)],
                                  nm1_ref.at[pl.ds(yi, 1)], sem).wait()
            pltpu.make_async_copy(u2_ref.at[pl.ds(i, 1)],
                                  nm2_ref.at[pl.ds(yi, 1)], sem).wait()


def _patch(y, u1, u2, c1, c2):
    return pl.pallas_call(
        _patch_kernel,
        in_specs=[
            pl.BlockSpec(memory_space=_SMEM),
            pl.BlockSpec(memory_space=_VMEM),
            pl.BlockSpec(memory_space=_VMEM),
            pl.BlockSpec(memory_space=_HBM),
            pl.BlockSpec(memory_space=_HBM),
        ],
        out_specs=[pl.BlockSpec(memory_space=_HBM)] * 2,
        out_shape=[jax.ShapeDtypeStruct(c1.shape, c1.dtype)] * 2,
        input_output_aliases={3: 0, 4: 1},
        scratch_shapes=[pltpu.SemaphoreType.DMA],
    )(y, u1, u2, c1, c2)


def kernel(v1, v2, y, idx, memory_v1, memory_v2):
    b, d = v1.shape
    n = memory_v1.shape[0]
    kk = idx.shape[1]
    total = b * kk

    w = _sc_gather(memory_v1, memory_v2, idx.reshape(total), total)
    w4 = w.reshape(2, b, kk, d)

    part, c1, c2t, anchors, u1, u2 = _pass1(w4, y, v1, v2,
                                            memory_v1, memory_v2)
    zmul = float(n) / float(b * b * kk * d)
    out_v1, c2h = _pass2(0, w4, anchors, part, zmul, memory_v2, c2t)
    out_v2, c2 = _pass2(1, w4, anchors, part, zmul, memory_v2, c2h)
    new_m1, new_m2 = _patch(y, u1, u2, c1, c2)
    return (out_v1, out_v2, new_m1, new_m2)
